# baseline (speedup 1.0000x reference)
import functools

import numpy as np
import jax
import jax.numpy as jnp
from jax import lax
from jax.experimental import pallas as pl
from jax.experimental.pallas import tpu as pltpu
from jax.experimental.pallas import tpu_sc as plsc

_D_MODEL = 256
_NUM_CLASSES = 91
_NUM_DN_GROUPS = 5
_BOX_NOISE_SCALE = 0.4
_LABEL_NOISE_RATIO = 0.2
_LANES = 16


@functools.lru_cache(maxsize=None)
def _rng_consts(b, n_dn, num_classes):
    cpu = jax.devices("cpu")[0]
    with jax.ensure_compile_time_eval():
        with jax.default_device(cpu):
            nkey = jax.random.key(42)
            kn, km, kr = jax.random.split(nkey, 3)
            noise = jax.random.uniform(kn, (b, n_dn, 4), dtype=jnp.float32) * 2.0 - 1.0
            noise_mask = jax.random.uniform(km, (b, n_dn)) < _LABEL_NOISE_RATIO
            rand_labels = jax.random.randint(kr, (b, n_dn), 0, num_classes,
                                             dtype=jnp.int32)
    noise = np.asarray(noise)
    rand_or_neg = np.where(np.asarray(noise_mask), np.asarray(rand_labels),
                           np.int32(-1)).astype(np.int32)
    return (noise[..., 0], noise[..., 1], noise[..., 2], noise[..., 3],
            rand_or_neg)


@functools.lru_cache(maxsize=None)
def _build_sc_gather(b, n_dn, d):
    info = plsc.get_sparse_core_info()
    nc, ns = info.num_cores, info.num_subcores
    nw = nc * ns
    n_rows = b * n_dn
    per_w = n_rows // nw
    assert per_w * nw == n_rows and per_w % 8 == 0
    chunk = 80
    n_chunks = per_w // chunk
    assert n_chunks * chunk == per_w and chunk % _LANES == 0
    mesh = plsc.VectorSubcoreMesh(core_axis_name="c", subcore_axis_name="s")

    scratch = [
        pltpu.VMEM((per_w,), jnp.int32),
        pltpu.VMEM((per_w,), jnp.int32),
        pltpu.VMEM((n_chunks, chunk), jnp.int32),
        pltpu.VMEM((per_w, d), jnp.float32),
        [pltpu.SemaphoreType.DMA] * n_chunks,
        pltpu.SemaphoreType.DMA,
    ]

    @functools.partial(
        pl.kernel,
        out_type=jax.ShapeDtypeStruct((n_rows, d), jnp.float32),
        mesh=mesh,
        scratch_types=scratch,
    )
    def sc_gather(lab_hbm, rnd_hbm, table_hbm, out_hbm,
                  lab_v, rnd_v, sel_v, rows_v, gsems, wsem):
        wid = lax.axis_index("s") * nc + lax.axis_index("c")
        base = wid * per_w
        pltpu.sync_copy(lab_hbm.at[pl.ds(base, per_w)], lab_v)
        pltpu.sync_copy(rnd_hbm.at[pl.ds(base, per_w)], rnd_v)
        for k in range(n_chunks):
            for o in range(0, chunk, _LANES):
                sl = pl.ds(k * chunk + o, _LANES)
                r = rnd_v[sl]
                sel_v[k, pl.ds(o, _LANES)] = jnp.where(r >= 0, r, lab_v[sl])
        gathers = []
        for k in range(n_chunks):
            cp = pltpu.make_async_copy(
                table_hbm.at[sel_v.at[k]],
                rows_v.at[pl.ds(k * chunk, chunk)],
                gsems[k],
            )
            cp.start()
            gathers.append(cp)
        for cp in gathers:
            cp.wait()
        pltpu.sync_copy(rows_v, out_hbm.at[pl.ds(base, per_w)])
        del wsem

    return sc_gather


def _retile_kernel(flat, b, n_dn, d):
    b_blk = 4

    def body(x_ref, o_ref):
        o_ref[...] = x_ref[...].reshape(b_blk, n_dn, d)

    return pl.pallas_call(
        body,
        grid=(b // b_blk,),
        in_specs=[pl.BlockSpec((b_blk * n_dn, d), lambda i: (i, 0))],
        out_specs=pl.BlockSpec((b_blk, n_dn, d), lambda i: (i, 0, 0)),
        out_shape=jax.ShapeDtypeStruct((b, n_dn, d), jnp.float32),
    )(flat)


def _mask_kernel(total_q, n_dn, max_gt, boundary):
    rows_per_block = 200
    grid = total_q // rows_per_block

    def body(bnd_ref, o_ref):
        row0 = pl.program_id(0) * rows_per_block
        bnd = bnd_ref[0]
        i = lax.broadcasted_iota(jnp.int32, (rows_per_block, total_q), 0) + row0
        j = lax.broadcasted_iota(jnp.int32, (rows_per_block, total_q), 1)
        base = (i >= bnd) & (j < bnd)
        extra = (i < n_dn) & (j < n_dn) & ((i // max_gt) != (j // max_gt))
        o_ref[...] = base | extra

    return pl.pallas_call(
        body,
        grid=(grid,),
        in_specs=[pl.BlockSpec(memory_space=pltpu.SMEM)],
        out_specs=pl.BlockSpec((rows_per_block, total_q), lambda i: (i, 0)),
        out_shape=jax.ShapeDtypeStruct((total_q, total_q), jnp.bool_),
    )(boundary)


def _box_noise_kernel(cx, cy, w, h, n0, n1, n2, n3):
    def body(cx_r, cy_r, w_r, h_r, n0_r, n1_r, n2_r, n3_r,
             ocx, ocy, ow, oh):
        wv = w_r[...]
        hv = h_r[...]
        s = _BOX_NOISE_SCALE
        ocx[...] = jnp.clip(cx_r[...] + n0_r[...] * (wv / 2.0) * s, 0.0, 1.0)
        ocy[...] = jnp.clip(cy_r[...] + n1_r[...] * (hv / 2.0) * s, 0.0, 1.0)
        ow[...] = jnp.clip(wv + n2_r[...] * wv * s, 0.0, 1.0)
        oh[...] = jnp.clip(hv + n3_r[...] * hv * s, 0.0, 1.0)

    shape = jax.ShapeDtypeStruct(cx.shape, jnp.float32)
    return pl.pallas_call(
        body,
        out_shape=(shape, shape, shape, shape),
    )(cx, cy, w, h, n0, n1, n2, n3)


def kernel(gt_boxes, gt_labels, num_queries, label_embed):
    b, max_gt = gt_labels.shape
    num_classes, d_model = label_embed.shape
    max_dn = 100
    eff_groups = min(_NUM_DN_GROUPS, max(1, max_dn // max_gt))
    n_dn = max_gt * eff_groups
    total_q = n_dn + 900

    boxes_rep = jnp.tile(gt_boxes, (1, eff_groups, 1))
    labels_rep = jnp.tile(gt_labels, (1, eff_groups))

    n0, n1, n2, n3, rand_or_neg = _rng_consts(b, n_dn, num_classes)

    boundary = jnp.asarray(total_q - num_queries, jnp.int32).reshape(1)
    attn_mask = _mask_kernel(total_q, n_dn, max_gt, boundary)

    ocx, ocy, ow, oh = _box_noise_kernel(
        boxes_rep[..., 0], boxes_rep[..., 1],
        boxes_rep[..., 2], boxes_rep[..., 3],
        jnp.asarray(n0), jnp.asarray(n1), jnp.asarray(n2), jnp.asarray(n3),
    )
    dn_reference_points = jnp.stack([ocx, ocy, ow, oh], axis=-1)

    sc_gather = _build_sc_gather(b, n_dn, d_model)
    dn_flat = sc_gather(labels_rep.reshape(-1),
                        jnp.asarray(rand_or_neg).reshape(-1),
                        label_embed)
    dn_queries = _retile_kernel(dn_flat, b, n_dn, d_model)

    return (dn_queries, dn_reference_points, labels_rep, boxes_rep, attn_mask)


# speedup vs baseline: 1.0840x; 1.0840x over previous
import functools

import numpy as np
import jax
import jax.numpy as jnp
from jax import lax
from jax.experimental import pallas as pl
from jax.experimental.pallas import tpu as pltpu
from jax.experimental.pallas import tpu_sc as plsc

_D_MODEL = 256
_NUM_CLASSES = 91
_NUM_DN_GROUPS = 5
_BOX_NOISE_SCALE = 0.4
_LABEL_NOISE_RATIO = 0.2
_LANES = 16


@functools.lru_cache(maxsize=None)
def _rng_consts(b, n_dn, num_classes):
    cpu = jax.devices("cpu")[0]
    with jax.ensure_compile_time_eval():
        with jax.default_device(cpu):
            nkey = jax.random.key(42)
            kn, km, kr = jax.random.split(nkey, 3)
            noise = jax.random.uniform(kn, (b, n_dn, 4), dtype=jnp.float32) * 2.0 - 1.0
            noise_mask = jax.random.uniform(km, (b, n_dn)) < _LABEL_NOISE_RATIO
            rand_labels = jax.random.randint(kr, (b, n_dn), 0, num_classes,
                                             dtype=jnp.int32)
    noise = np.asarray(noise)
    rand_or_neg = np.where(np.asarray(noise_mask), np.asarray(rand_labels),
                           np.int32(-1)).astype(np.int32)
    return (noise[..., 0], noise[..., 1], noise[..., 2], noise[..., 3],
            rand_or_neg)


@functools.lru_cache(maxsize=None)
def _build_sc_gather(b, n_dn, d):
    info = plsc.get_sparse_core_info()
    nc, ns = info.num_cores, info.num_subcores
    nw = nc * ns
    n_rows = b * n_dn
    per_w = n_rows // nw
    assert per_w * nw == n_rows and per_w % 8 == 0
    chunk = 80
    n_chunks = per_w // chunk
    assert n_chunks * chunk == per_w and chunk % _LANES == 0
    mesh = plsc.VectorSubcoreMesh(core_axis_name="c", subcore_axis_name="s")

    scratch = [
        pltpu.VMEM((per_w,), jnp.int32),
        pltpu.VMEM((per_w,), jnp.int32),
        pltpu.VMEM((n_chunks, chunk), jnp.int32),
        pltpu.VMEM((per_w, d), jnp.float32),
        pltpu.SemaphoreType.DMA,
    ]

    @functools.partial(
        pl.kernel,
        out_type=jax.ShapeDtypeStruct((n_rows, d), jnp.float32),
        mesh=mesh,
        scratch_types=scratch,
    )
    def sc_gather(lab_hbm, rnd_hbm, table_hbm, out_hbm,
                  lab_v, rnd_v, sel_v, rows_v, sem):
        wid = lax.axis_index("s") * nc + lax.axis_index("c")
        base = wid * per_w
        pltpu.sync_copy(lab_hbm.at[pl.ds(base, per_w)], lab_v)
        pltpu.sync_copy(rnd_hbm.at[pl.ds(base, per_w)], rnd_v)
        for k in range(n_chunks):
            for o in range(0, chunk, _LANES):
                sl = pl.ds(k * chunk + o, _LANES)
                r = rnd_v[sl]
                sel_v[k, pl.ds(o, _LANES)] = jnp.where(r >= 0, r, lab_v[sl])
        gathers = []
        for k in range(n_chunks):
            cp = pltpu.make_async_copy(
                table_hbm.at[sel_v.at[k]],
                rows_v.at[pl.ds(k * chunk, chunk)],
                sem,
            )
            cp.start()
            gathers.append(cp)
        for cp in gathers:
            cp.wait()
        pltpu.sync_copy(rows_v, out_hbm.at[pl.ds(base, per_w)])

    return sc_gather


def _mask_kernel(total_q, n_dn, max_gt, boundary):
    rows_per_block = 200
    grid = total_q // rows_per_block

    def body(bnd_ref, o_ref):
        row0 = pl.program_id(0) * rows_per_block
        bnd = bnd_ref[0]
        i = lax.broadcasted_iota(jnp.int32, (rows_per_block, total_q), 0) + row0
        j = lax.broadcasted_iota(jnp.int32, (rows_per_block, total_q), 1)
        base = (i >= bnd) & (j < bnd)
        extra = (i < n_dn) & (j < n_dn) & ((i // max_gt) != (j // max_gt))
        o_ref[...] = base | extra

    return pl.pallas_call(
        body,
        grid=(grid,),
        in_specs=[pl.BlockSpec(memory_space=pltpu.SMEM)],
        out_specs=pl.BlockSpec((rows_per_block, total_q), lambda i: (i, 0)),
        out_shape=jax.ShapeDtypeStruct((total_q, total_q), jnp.bool_),
    )(boundary)


def _box_noise_kernel(cx, cy, w, h, n0, n1, n2, n3):
    def body(cx_r, cy_r, w_r, h_r, n0_r, n1_r, n2_r, n3_r,
             ocx, ocy, ow, oh):
        wv = w_r[...]
        hv = h_r[...]
        s = _BOX_NOISE_SCALE
        ocx[...] = jnp.clip(cx_r[...] + n0_r[...] * (wv / 2.0) * s, 0.0, 1.0)
        ocy[...] = jnp.clip(cy_r[...] + n1_r[...] * (hv / 2.0) * s, 0.0, 1.0)
        ow[...] = jnp.clip(wv + n2_r[...] * wv * s, 0.0, 1.0)
        oh[...] = jnp.clip(hv + n3_r[...] * hv * s, 0.0, 1.0)

    shape = jax.ShapeDtypeStruct(cx.shape, jnp.float32)
    return pl.pallas_call(
        body,
        out_shape=(shape, shape, shape, shape),
    )(cx, cy, w, h, n0, n1, n2, n3)


def kernel(gt_boxes, gt_labels, num_queries, label_embed):
    b, max_gt = gt_labels.shape
    num_classes, d_model = label_embed.shape
    max_dn = 100
    eff_groups = min(_NUM_DN_GROUPS, max(1, max_dn // max_gt))
    n_dn = max_gt * eff_groups
    total_q = n_dn + 900

    boxes_rep = jnp.tile(gt_boxes, (1, eff_groups, 1))
    labels_rep = jnp.tile(gt_labels, (1, eff_groups))

    n0, n1, n2, n3, rand_or_neg = _rng_consts(b, n_dn, num_classes)

    boundary = jnp.asarray(total_q - num_queries, jnp.int32).reshape(1)
    attn_mask = _mask_kernel(total_q, n_dn, max_gt, boundary)

    ocx, ocy, ow, oh = _box_noise_kernel(
        boxes_rep[..., 0], boxes_rep[..., 1],
        boxes_rep[..., 2], boxes_rep[..., 3],
        jnp.asarray(n0), jnp.asarray(n1), jnp.asarray(n2), jnp.asarray(n3),
    )
    dn_reference_points = jnp.stack([ocx, ocy, ow, oh], axis=-1)

    sc_gather = _build_sc_gather(b, n_dn, d_model)
    dn_queries = sc_gather(labels_rep.reshape(-1),
                           jnp.asarray(rand_or_neg).reshape(-1),
                           label_embed)
    dn_queries = dn_queries.reshape(b, n_dn, d_model)

    return (dn_queries, dn_reference_points, labels_rep, boxes_rep, attn_mask)


# speedup vs baseline: 1.0842x; 1.0001x over previous
import functools

import numpy as np
import jax
import jax.numpy as jnp
from jax import lax
from jax.experimental import pallas as pl
from jax.experimental.pallas import tpu as pltpu
from jax.experimental.pallas import tpu_sc as plsc

_NUM_DN_GROUPS = 5
_BOX_NOISE_SCALE = 0.4
_LABEL_NOISE_RATIO = 0.2
_LANES = 16


@functools.lru_cache(maxsize=None)
def _rng_consts(b, n_dn, num_classes):
    cpu = jax.devices("cpu")[0]
    with jax.ensure_compile_time_eval():
        with jax.default_device(cpu):
            nkey = jax.random.key(42)
            kn, km, kr = jax.random.split(nkey, 3)
            noise = jax.random.uniform(kn, (b, n_dn, 4), dtype=jnp.float32) * 2.0 - 1.0
            noise_mask = jax.random.uniform(km, (b, n_dn)) < _LABEL_NOISE_RATIO
            rand_labels = jax.random.randint(kr, (b, n_dn), 0, num_classes,
                                             dtype=jnp.int32)
    noise = np.asarray(noise)
    rand_or_neg = np.where(np.asarray(noise_mask), np.asarray(rand_labels),
                           np.int32(-1)).astype(np.int32)
    return (noise[..., 0], noise[..., 1], noise[..., 2], noise[..., 3],
            rand_or_neg)


@functools.lru_cache(maxsize=None)
def _build_sc_gather(b, n_dn, d):
    info = plsc.get_sparse_core_info()
    nc, ns = info.num_cores, info.num_subcores
    nw = nc * ns
    n_rows = b * n_dn
    per_w = n_rows // nw
    assert per_w * nw == n_rows and per_w % 8 == 0
    chunk = 80
    n_chunks = per_w // chunk
    assert n_chunks * chunk == per_w and chunk % _LANES == 0
    mesh = plsc.VectorSubcoreMesh(core_axis_name="c", subcore_axis_name="s")

    scratch = [
        pltpu.VMEM((per_w,), jnp.int32),
        pltpu.VMEM((per_w,), jnp.int32),
        pltpu.VMEM((n_chunks, chunk), jnp.int32),
        pltpu.VMEM((per_w, d), jnp.float32),
        pltpu.SemaphoreType.DMA,
    ]

    @functools.partial(
        pl.kernel,
        out_type=jax.ShapeDtypeStruct((n_rows, d), jnp.float32),
        mesh=mesh,
        scratch_types=scratch,
    )
    def sc_gather(lab_hbm, rnd_hbm, table_hbm, out_hbm,
                  lab_v, rnd_v, sel_v, rows_v, sem):
        wid = lax.axis_index("s") * nc + lax.axis_index("c")
        base = wid * per_w
        pltpu.sync_copy(lab_hbm.at[pl.ds(base, per_w)], lab_v)
        pltpu.sync_copy(rnd_hbm.at[pl.ds(base, per_w)], rnd_v)
        for k in range(n_chunks):
            for o in range(0, chunk, _LANES):
                sl = pl.ds(k * chunk + o, _LANES)
                r = rnd_v[sl]
                sel_v[k, pl.ds(o, _LANES)] = jnp.where(r >= 0, r, lab_v[sl])
        gathers = []
        for k in range(n_chunks):
            cp = pltpu.make_async_copy(
                table_hbm.at[sel_v.at[k]],
                rows_v.at[pl.ds(k * chunk, chunk)],
                sem,
            )
            cp.start()
            gathers.append(cp)
        for cp in gathers:
            cp.wait()
        pltpu.sync_copy(rows_v, out_hbm.at[pl.ds(base, per_w)])

    return sc_gather


def _mask_kernel(total_q, n_dn, max_gt, boundary):
    rows_per_block = 200
    grid = total_q // rows_per_block

    def body(bnd_ref, o_ref):
        row0 = pl.program_id(0) * rows_per_block
        bnd = bnd_ref[0]
        i = lax.broadcasted_iota(jnp.int32, (rows_per_block, total_q), 0) + row0
        j = lax.broadcasted_iota(jnp.int32, (rows_per_block, total_q), 1)
        base = (i >= bnd) & (j < bnd)
        extra = (i < n_dn) & (j < n_dn) & ((i // max_gt) != (j // max_gt))
        o_ref[...] = base | extra

    return pl.pallas_call(
        body,
        grid=(grid,),
        in_specs=[pl.BlockSpec(memory_space=pltpu.SMEM)],
        out_specs=pl.BlockSpec((rows_per_block, total_q), lambda i: (i, 0)),
        out_shape=jax.ShapeDtypeStruct((total_q, total_q), jnp.bool_),
    )(boundary)


def _box_noise_kernel(cx, cy, w, h, n0, n1, n2, n3):
    def body(cx_r, cy_r, w_r, h_r, n0_r, n1_r, n2_r, n3_r,
             ocx, ocy, ow, oh):
        wv = w_r[...]
        hv = h_r[...]
        s = _BOX_NOISE_SCALE
        ocx[...] = jnp.clip(cx_r[...] + n0_r[...] * (wv / 2.0) * s, 0.0, 1.0)
        ocy[...] = jnp.clip(cy_r[...] + n1_r[...] * (hv / 2.0) * s, 0.0, 1.0)
        ow[...] = jnp.clip(wv + n2_r[...] * wv * s, 0.0, 1.0)
        oh[...] = jnp.clip(hv + n3_r[...] * hv * s, 0.0, 1.0)

    shape = jax.ShapeDtypeStruct(cx.shape, jnp.float32)
    return pl.pallas_call(
        body,
        out_shape=(shape, shape, shape, shape),
    )(cx, cy, w, h, n0, n1, n2, n3)


def kernel(gt_boxes, gt_labels, num_queries, label_embed):
    b, max_gt = gt_labels.shape
    num_classes, d_model = label_embed.shape
    max_dn = 100
    eff_groups = min(_NUM_DN_GROUPS, max(1, max_dn // max_gt))
    n_dn = max_gt * eff_groups
    total_q = n_dn + 900

    boxes_rep = jnp.tile(gt_boxes, (1, eff_groups, 1))
    labels_rep = jnp.tile(gt_labels, (1, eff_groups))

    n0, n1, n2, n3, rand_or_neg = _rng_consts(b, n_dn, num_classes)

    boundary = jnp.asarray(total_q - num_queries, jnp.int32).reshape(1)
    attn_mask = _mask_kernel(total_q, n_dn, max_gt, boundary)

    ocx, ocy, ow, oh = _box_noise_kernel(
        boxes_rep[..., 0], boxes_rep[..., 1],
        boxes_rep[..., 2], boxes_rep[..., 3],
        jnp.asarray(n0), jnp.asarray(n1), jnp.asarray(n2), jnp.asarray(n3),
    )
    dn_reference_points = jnp.stack([ocx, ocy, ow, oh], axis=-1)

    sc_gather = _build_sc_gather(b, n_dn, d_model)
    dn_queries = sc_gather(labels_rep.reshape(-1),
                           jnp.asarray(rand_or_neg).reshape(-1),
                           label_embed)
    dn_queries = dn_queries.reshape(b, n_dn, d_model)

    return (dn_queries, dn_reference_points, labels_rep, boxes_rep, attn_mask)
